# Initial kernel scaffold; baseline (speedup 1.0000x reference)
import jax
import jax.numpy as jnp
from jax.experimental import pallas as pl


def kernel(x, edge_index, adjacency, W_enc, b_enc, W1, b1, skip1, W2, b2, skip2, Wa, ba):
    raise NotImplementedError("write your pallas kernel here")


# speedup vs baseline: 3.4893x; 3.4893x over previous
import functools

import jax
import jax.numpy as jnp
from jax import lax
from jax.experimental import pallas as pl
from jax.experimental.pallas import tpu as pltpu
from jax.experimental.pallas import tpu_sc as plsc

N = 10000
D_FEAT = 128
D_EXT = 144
HIDDEN = 64
N_CLUSTERS = 16

_NC = 2
_NS = 16
_NW = _NC * _NS

_E = 320000
_EDGES_PER_W = _E // _NW
_CHUNK = 80
_NCHUNK = _EDGES_PER_W // _CHUNK
_N_PAD = 10240
_ROWS_PER_S = _N_PAD // _NS

_SELU_ALPHA = 1.6732632423543772
_SELU_SCALE = 1.0507009873554805


def _selu(x):
    return _SELU_SCALE * jnp.where(
        x > 0, x, _SELU_ALPHA * (jnp.exp(jnp.minimum(x, 0.0)) - 1.0))


def _enc_body(x_ref, w_ref, b_ref, out_ref):
    h = jnp.dot(x_ref[...], w_ref[...], preferred_element_type=jnp.float32)
    h = h + b_ref[...]
    out_ref[...] = jnp.concatenate(
        [h, jnp.ones((h.shape[0], D_EXT - D_FEAT), jnp.float32)], axis=1)


def _enc(x, w, b):
    r = 1000
    return pl.pallas_call(
        _enc_body,
        grid=(N // r,),
        in_specs=[
            pl.BlockSpec((r, D_FEAT), lambda i: (i, 0)),
            pl.BlockSpec((D_FEAT, D_FEAT), lambda i: (0, 0)),
            pl.BlockSpec((1, D_FEAT), lambda i: (0, 0)),
        ],
        out_specs=pl.BlockSpec((r, D_EXT), lambda i: (i, 0)),
        out_shape=jax.ShapeDtypeStruct((N, D_EXT), jnp.float32),
    )(x, w, b)


def _sc_body(h_hbm, src_hbm, dst_hbm, z_hbm, out_hbm,
             src_t, dst_t, rows_v, acc_sh, sem):
    c = lax.axis_index("c")
    s = lax.axis_index("s")
    wid = s * _NC + c

    pltpu.sync_copy(src_hbm.at[wid], src_t)
    pltpu.sync_copy(dst_hbm.at[wid], dst_t)
    pltpu.sync_copy(z_hbm, acc_sh.at[pl.ds(s * _ROWS_PER_S, _ROWS_PER_S)])
    plsc.subcore_barrier()

    def chunk(ch, carry):
        pltpu.async_copy(h_hbm.at[src_t.at[ch]], rows_v, sem).wait()
        pltpu.sync_copy(rows_v, acc_sh.at[dst_t.at[ch]], add=True)
        return carry

    lax.fori_loop(0, _NCHUNK, chunk, 0)
    plsc.subcore_barrier()
    pltpu.sync_copy(acc_sh.at[pl.ds(s * _ROWS_PER_S, _ROWS_PER_S)],
                    out_hbm.at[c, pl.ds(s * _ROWS_PER_S, _ROWS_PER_S)])


def _sc_segsum(h_ext, src_r, dst_r, zeros_stripe):
    mesh = plsc.VectorSubcoreMesh(
        core_axis_name="c", subcore_axis_name="s",
        num_cores=_NC, num_subcores=_NS)
    k = pl.kernel(
        _sc_body,
        out_type=jax.ShapeDtypeStruct((_NC, _N_PAD, D_EXT), jnp.float32),
        mesh=mesh,
        scratch_types=[
            pltpu.VMEM((_NCHUNK, _CHUNK), jnp.int32),
            pltpu.VMEM((_NCHUNK, _CHUNK), jnp.int32),
            pltpu.VMEM((_CHUNK, D_EXT), jnp.float32),
            pltpu.VMEM_SHARED((_N_PAD, D_EXT), jnp.float32),
            pltpu.SemaphoreType.DMA,
        ],
        compiler_params=pltpu.CompilerParams(use_tc_tiling_on_sc=False),
    )
    return k(h_ext, src_r, dst_r, zeros_stripe)


def _mid_body(h_ref, acc_ref, w1_ref, b1_ref, out_ref):
    acc = acc_ref[0] + acc_ref[1]
    deg = jnp.maximum(acc[:, D_FEAT:D_FEAT + 1], 1.0)
    h = jnp.maximum(h_ref[:, :D_FEAT] + acc[:, :D_FEAT] / deg, 0.0)
    out_ref[...] = (
        jnp.dot(h, w1_ref[...], preferred_element_type=jnp.float32)
        + b1_ref[...])


def _mid(h_ext, acc2, w1, b1):
    r = 1000
    return pl.pallas_call(
        _mid_body,
        grid=(N // r,),
        in_specs=[
            pl.BlockSpec((r, D_EXT), lambda i: (i, 0)),
            pl.BlockSpec((_NC, r, D_EXT), lambda i: (0, i, 0)),
            pl.BlockSpec((D_FEAT, HIDDEN), lambda i: (0, 0)),
            pl.BlockSpec((1, HIDDEN), lambda i: (0, 0)),
        ],
        out_specs=pl.BlockSpec((r, HIDDEN), lambda i: (i, 0)),
        out_shape=jax.ShapeDtypeStruct((N, HIDDEN), jnp.float32),
    )(h_ext, acc2, w1, b1)


def _gcn1_body(a_ref, t1_ref, t1row_ref, skip_ref, w2_ref, b2_ref, out_ref):
    p = jnp.dot(a_ref[...], t1_ref[...], preferred_element_type=jnp.float32)
    h1 = _selu(skip_ref[...] * t1row_ref[...] + p)
    out_ref[...] = (
        jnp.dot(h1, w2_ref[...], preferred_element_type=jnp.float32)
        + b2_ref[...])


def _gcn1(adj, t1, skip1, w2, b2, r):
    return pl.pallas_call(
        _gcn1_body,
        grid=(N // r,),
        in_specs=[
            pl.BlockSpec((r, N), lambda i: (i, 0)),
            pl.BlockSpec((N, HIDDEN), lambda i: (0, 0)),
            pl.BlockSpec((r, HIDDEN), lambda i: (i, 0)),
            pl.BlockSpec((1, HIDDEN), lambda i: (0, 0)),
            pl.BlockSpec((HIDDEN, HIDDEN), lambda i: (0, 0)),
            pl.BlockSpec((1, HIDDEN), lambda i: (0, 0)),
        ],
        out_specs=pl.BlockSpec((r, HIDDEN), lambda i: (i, 0)),
        out_shape=jax.ShapeDtypeStruct((N, HIDDEN), jnp.float32),
    )(adj, t1, t1, skip1, w2, b2)


def _gcn2_body(a_ref, t2_ref, t2row_ref, skip_ref, wa_ref, ba_ref, out_ref):
    p = jnp.dot(a_ref[...], t2_ref[...], preferred_element_type=jnp.float32)
    h2 = _selu(skip_ref[...] * t2row_ref[...] + p)
    logits = (
        jnp.dot(h2, wa_ref[...], preferred_element_type=jnp.float32)
        + ba_ref[...])
    m = jnp.max(logits, axis=-1, keepdims=True)
    e = jnp.exp(logits - m)
    out_ref[...] = e / jnp.sum(e, axis=-1, keepdims=True)


def _gcn2(adj, t2, skip2, wa, ba, r):
    return pl.pallas_call(
        _gcn2_body,
        grid=(N // r,),
        in_specs=[
            pl.BlockSpec((r, N), lambda i: (i, 0)),
            pl.BlockSpec((N, HIDDEN), lambda i: (0, 0)),
            pl.BlockSpec((r, HIDDEN), lambda i: (i, 0)),
            pl.BlockSpec((1, HIDDEN), lambda i: (0, 0)),
            pl.BlockSpec((HIDDEN, N_CLUSTERS), lambda i: (0, 0)),
            pl.BlockSpec((1, N_CLUSTERS), lambda i: (0, 0)),
        ],
        out_specs=pl.BlockSpec((r, N_CLUSTERS), lambda i: (i, 0)),
        out_shape=jax.ShapeDtypeStruct((N, N_CLUSTERS), jnp.float32),
    )(adj, t2, t2, skip2, wa, ba)


def kernel(x, edge_index, adjacency, W_enc, b_enc, W1, b1, skip1,
           W2, b2, skip2, Wa, ba):
    src_r = edge_index[0].astype(jnp.int32).reshape(_NW, _NCHUNK, _CHUNK)
    dst_r = edge_index[1].astype(jnp.int32).reshape(_NW, _NCHUNK, _CHUNK)
    zeros_stripe = jnp.zeros((_ROWS_PER_S, D_EXT), jnp.float32)

    h_ext = _enc(x, W_enc, b_enc.reshape(1, D_FEAT))
    acc2 = _sc_segsum(h_ext, src_r, dst_r, zeros_stripe)
    t1 = _mid(h_ext, acc2, W1, b1.reshape(1, HIDDEN))
    t2 = _gcn1(adjacency, t1, skip1.reshape(1, HIDDEN),
               W2, b2.reshape(1, HIDDEN), r=80)
    return _gcn2(adjacency, t2, skip2.reshape(1, HIDDEN),
                 Wa, ba.reshape(1, N_CLUSTERS), r=80)
